# baseline (speedup 1.0000x reference)
import functools

import jax
import jax.numpy as jnp
from jax import lax
from jax.experimental import pallas as pl
from jax.experimental.pallas import tpu as pltpu
from jax.experimental.pallas import tpu_sc as plsc

N = 16384
D = 64
_NSEA = 1000

_NC = 2
_NS = 16
_NW = _NC * _NS
_RPW = N // _NW
_CHUNK = 64
_NCH = _RPW // _CHUNK
_AUX = D * 16 + _NSEA + 24

_mesh = plsc.VectorSubcoreMesh(core_axis_name="c", subcore_axis_name="s")


@functools.partial(
    pl.kernel,
    mesh=_mesh,
    compiler_params=pltpu.CompilerParams(needs_layout_passes=False),
    out_type=jax.ShapeDtypeStruct((N,), jnp.float32),
    scratch_types=[
        pltpu.VMEM((3, _RPW), jnp.int32),
        pltpu.VMEM((D, _RPW), jnp.float32),
        pltpu.VMEM((_AUX,), jnp.float32),
        pltpu.VMEM((_RPW,), jnp.float32),
        pltpu.VMEM((_RPW,), jnp.float32),
        pltpu.VMEM((_RPW,), jnp.float32),
        pltpu.SemaphoreType.DMA,
        pltpu.SemaphoreType.DMA,
        pltpu.SemaphoreType.DMA,
    ],
)
def _sc_fused(xt_hbm, ids3_hbm, aux_hbm, u_hbm, v_hbm, out_hbm,
              idb, xcol, aux, acc, gu, gv, sem_i, sem_x, sem_g):
    wid = lax.axis_index("s") * _NC + lax.axis_index("c")
    base = wid * _RPW

    icp = pltpu.async_copy(ids3_hbm.at[:, pl.ds(base, _RPW)], idb, sem_i)
    acp = pltpu.async_copy(aux_hbm, aux, sem_i)
    xcp = pltpu.async_copy(xt_hbm.at[:, pl.ds(base, _RPW)], xcol, sem_x)
    icp.wait()
    acp.wait()

    gathers = []
    for j in range(_NCH):
        sl = pl.ds(j * _CHUNK, _CHUNK)
        gathers.append(
            pltpu.async_copy(u_hbm.at[idb.at[0, sl]], gu.at[sl], sem_g))
        gathers.append(
            pltpu.async_copy(v_hbm.at[idb.at[1, sl]], gv.at[sl], sem_g))

    xcp.wait()

    def chunk_body(c, _):
        r = pl.ds(c * 16, 16)
        a = xcol[0, r] * aux[pl.ds(0, 16)]
        for d in range(1, D):
            a = a + xcol[d, r] * aux[pl.ds(d * 16, 16)]
        acc[r] = a
        return _

    lax.fori_loop(0, _RPW // 16, chunk_body, 0)

    for i in range(_RPW // 16):
        s = pl.ds(i * 16, 16)
        acc[s] = acc[s] + plsc.load_gather(aux, [idb[2, s] + D * 16])

    for c in gathers:
        c.wait()

    for i in range(_RPW // 16):
        s = pl.ds(i * 16, 16)
        acc[s] = acc[s] + gu[s] + gv[s]

    pltpu.sync_copy(acc, out_hbm.at[pl.ds(base, _RPW)])


def kernel(X, pro_id, celeb_id, season, beta, u_pro, v_celeb, w_season):
    ids3 = jnp.stack([pro_id.astype(jnp.int32), celeb_id.astype(jnp.int32),
                      season.astype(jnp.int32)])
    aux = jnp.concatenate(
        [jnp.repeat(beta, 16), w_season,
         jnp.zeros((_AUX - D * 16 - _NSEA,), jnp.float32)])
    return _sc_fused(X.T, ids3, aux, u_pro, v_celeb)


# speedup vs baseline: 1.0438x; 1.0438x over previous
import functools

import jax
import jax.numpy as jnp
from jax import lax
from jax.experimental import pallas as pl
from jax.experimental.pallas import tpu as pltpu
from jax.experimental.pallas import tpu_sc as plsc

N = 16384
D = 64

_NC = 2
_NS = 16
_NW = _NC * _NS
_RPW = N // _NW
_CHUNK = 64
_NCH = _RPW // _CHUNK

_mesh = plsc.VectorSubcoreMesh(core_axis_name="c", subcore_axis_name="s")


@functools.partial(
    pl.kernel,
    mesh=_mesh,
    compiler_params=pltpu.CompilerParams(needs_layout_passes=False),
    out_type=jax.ShapeDtypeStruct((N,), jnp.float32),
    scratch_types=[
        pltpu.VMEM((_RPW,), jnp.int32),
        pltpu.VMEM((_RPW,), jnp.int32),
        pltpu.VMEM((_RPW,), jnp.int32),
        pltpu.VMEM((D, _RPW), jnp.float32),
        pltpu.VMEM((D * 16,), jnp.float32),
        pltpu.VMEM((_RPW,), jnp.float32),
        pltpu.VMEM((_RPW,), jnp.float32),
        pltpu.VMEM((_RPW,), jnp.float32),
        pltpu.VMEM((1024,), jnp.float32),
        pltpu.SemaphoreType.DMA,
        pltpu.SemaphoreType.DMA,
        pltpu.SemaphoreType.DMA,
    ],
)
def _sc_fused(xt_hbm, pro_hbm, celeb_hbm, season_hbm, beta_hbm, u_hbm, v_hbm,
              w_hbm, out_hbm, idu, idv, ids, xcol, bbv, acc, gu, gv, wtab,
              sem_i, sem_x, sem_g):
    wid = lax.axis_index("s") * _NC + lax.axis_index("c")
    base = wid * _RPW

    stage = [
        pltpu.async_copy(pro_hbm.at[pl.ds(base, _RPW)], idu, sem_i),
        pltpu.async_copy(celeb_hbm.at[pl.ds(base, _RPW)], idv, sem_i),
        pltpu.async_copy(season_hbm.at[pl.ds(base, _RPW)], ids, sem_i),
        pltpu.async_copy(beta_hbm, bbv, sem_i),
        pltpu.async_copy(w_hbm, wtab.at[pl.ds(0, 1000)], sem_i),
    ]
    xcp = pltpu.async_copy(xt_hbm.at[:, pl.ds(base, _RPW)], xcol, sem_x)
    for c in stage:
        c.wait()

    gathers = []
    for j in range(_NCH):
        sl = pl.ds(j * _CHUNK, _CHUNK)
        gathers.append(pltpu.async_copy(u_hbm.at[idu.at[sl]], gu.at[sl], sem_g))
        gathers.append(pltpu.async_copy(v_hbm.at[idv.at[sl]], gv.at[sl], sem_g))

    xcp.wait()

    def chunk_body(c, _):
        r = pl.ds(c * 16, 16)
        a = xcol[0, r] * bbv[pl.ds(0, 16)]
        for d in range(1, D):
            a = a + xcol[d, r] * bbv[pl.ds(d * 16, 16)]
        acc[r] = a
        return _

    lax.fori_loop(0, _RPW // 16, chunk_body, 0)

    for c in gathers:
        c.wait()

    def add_body(i, _):
        s = pl.ds(i * 16, 16)
        acc[s] = acc[s] + gu[s] + gv[s] + plsc.load_gather(wtab, [ids[s]])
        return _

    lax.fori_loop(0, _RPW // 16, add_body, 0)

    pltpu.sync_copy(acc, out_hbm.at[pl.ds(base, _RPW)])


def kernel(X, pro_id, celeb_id, season, beta, u_pro, v_celeb, w_season):
    return _sc_fused(
        X.T,
        pro_id.astype(jnp.int32),
        celeb_id.astype(jnp.int32),
        season.astype(jnp.int32),
        jnp.repeat(beta, 16),
        u_pro,
        v_celeb,
        w_season,
    )


# speedup vs baseline: 1.0513x; 1.0072x over previous
import functools

import jax
import jax.numpy as jnp
from jax import lax
from jax.experimental import pallas as pl
from jax.experimental.pallas import tpu as pltpu
from jax.experimental.pallas import tpu_sc as plsc

N = 16384
D = 64

_NC = 2
_NS = 16
_NW = _NC * _NS
_RPW = N // _NW
_CHUNK = 64
_NCH = _RPW // _CHUNK

_mesh = plsc.VectorSubcoreMesh(core_axis_name="c", subcore_axis_name="s")


@functools.partial(
    pl.kernel,
    mesh=_mesh,
    compiler_params=pltpu.CompilerParams(needs_layout_passes=False),
    out_type=jax.ShapeDtypeStruct((N,), jnp.float32),
    scratch_types=[
        pltpu.VMEM((_RPW,), jnp.int32),
        pltpu.VMEM((_RPW,), jnp.int32),
        pltpu.VMEM((_RPW,), jnp.int32),
        pltpu.VMEM((D, _RPW), jnp.float32),
        pltpu.VMEM((D * 16,), jnp.float32),
        pltpu.VMEM((_RPW,), jnp.float32),
        pltpu.VMEM((_RPW,), jnp.float32),
        pltpu.VMEM((_RPW,), jnp.float32),
        pltpu.VMEM((1024,), jnp.float32),
        pltpu.SemaphoreType.DMA,
        pltpu.SemaphoreType.DMA,
        pltpu.SemaphoreType.DMA,
    ],
)
def _sc_fused(xt_hbm, pro_hbm, celeb_hbm, season_hbm, beta_hbm, u_hbm, v_hbm,
              w_hbm, out_hbm, idu, idv, ids, xcol, bbv, acc, gu, gv, wtab,
              sem_i, sem_x, sem_g):
    wid = lax.axis_index("s") * _NC + lax.axis_index("c")
    base = wid * _RPW

    stage = [
        pltpu.async_copy(pro_hbm.at[pl.ds(base, _RPW)], idu, sem_i),
        pltpu.async_copy(celeb_hbm.at[pl.ds(base, _RPW)], idv, sem_i),
        pltpu.async_copy(season_hbm.at[pl.ds(base, _RPW)], ids, sem_i),
        pltpu.async_copy(beta_hbm, bbv, sem_i),
        pltpu.async_copy(w_hbm, wtab.at[pl.ds(0, 1000)], sem_i),
    ]
    xcp = pltpu.async_copy(xt_hbm.at[:, pl.ds(base, _RPW)], xcol, sem_x)
    for c in stage:
        c.wait()

    def fire_body(j, _):
        sl = pl.ds(j * _CHUNK, _CHUNK)
        pltpu.async_copy(u_hbm.at[idu.at[sl]], gu.at[sl], sem_g)
        pltpu.async_copy(v_hbm.at[idv.at[sl]], gv.at[sl], sem_g)
        return _

    lax.fori_loop(0, _NCH, fire_body, 0)

    xcp.wait()

    def chunk_body(c, _):
        r = pl.ds(c * 16, 16)
        a = xcol[0, r] * bbv[pl.ds(0, 16)]
        for d in range(1, D):
            a = a + xcol[d, r] * bbv[pl.ds(d * 16, 16)]
        acc[r] = a
        return _

    lax.fori_loop(0, _RPW // 16, chunk_body, 0)

    pltpu.make_async_copy(u_hbm.at[pl.ds(0, _RPW)], gu, sem_g).wait()
    pltpu.make_async_copy(v_hbm.at[pl.ds(0, _RPW)], gv, sem_g).wait()

    def add_body(i, _):
        s = pl.ds(i * 16, 16)
        acc[s] = acc[s] + gu[s] + gv[s] + plsc.load_gather(wtab, [ids[s]])
        return _

    lax.fori_loop(0, _RPW // 16, add_body, 0)

    pltpu.sync_copy(acc, out_hbm.at[pl.ds(base, _RPW)])


def kernel(X, pro_id, celeb_id, season, beta, u_pro, v_celeb, w_season):
    return _sc_fused(
        X.T,
        pro_id.astype(jnp.int32),
        celeb_id.astype(jnp.int32),
        season.astype(jnp.int32),
        jnp.repeat(beta, 16),
        u_pro,
        v_celeb,
        w_season,
    )
